# baseline (speedup 1.0000x reference)
import jax
import jax.numpy as jnp
from jax import lax
from jax.experimental import pallas as pl
from jax.experimental.pallas import tpu as pltpu
from jax.experimental.pallas import tpu_sc as plsc

NC, NS = 2, 16
NW = NC * NS
ROWS = 16384
D = 1024
RPW = ROWS // NW
CHUNK = 32
NCHUNK = RPW // CHUNK
NBUF = 4
LEAD = NBUF - 2


def _sc_body(x2_hbm, out_hbm, tbuf, sbuf, *sems):
    sid = lax.axis_index("s")
    wid = sid * NC + lax.axis_index("c")
    mysbuf = sbuf.at[sid]
    b = wid // 8
    h = (wid % 8) // 4
    p = wid % 4
    src_row0 = b * 2048 + p * RPW
    src_col0 = h * D
    dst_row0 = wid * RPW

    gsems, psems = sems[:NBUF], sems[NBUF:]

    def slot(k):
        s = k % NBUF
        return tbuf.at[s] if s < 2 else mysbuf.at[s - 2]

    def gather(k):
        return pltpu.async_copy(
            x2_hbm.at[pl.ds(src_row0 + k * CHUNK, CHUNK), pl.ds(src_col0, D)],
            slot(k), gsems[k % NBUF])

    def put(k):
        return pltpu.async_copy(
            slot(k), out_hbm.at[pl.ds(dst_row0 + k * CHUNK, CHUNK)],
            psems[k % NBUF])

    gd = [None] * NCHUNK
    pd = [None] * NCHUNK
    for k in range(min(LEAD, NCHUNK)):
        gd[k] = gather(k)
    for k in range(NCHUNK):
        j = k + LEAD
        if j < NCHUNK:
            if j >= NBUF:
                pd[j - NBUF].wait()
            gd[j] = gather(j)
        gd[k].wait()
        pd[k] = put(k)
    for k in range(max(0, NCHUNK - NBUF), NCHUNK):
        pd[k].wait()


def _shuffle_sc(x2):
    mesh = plsc.VectorSubcoreMesh(core_axis_name="c", subcore_axis_name="s")
    return pl.kernel(
        _sc_body,
        out_type=jax.ShapeDtypeStruct((ROWS, D), jnp.float32),
        mesh=mesh,
        scratch_types=[
            pltpu.VMEM((2, CHUNK, D), jnp.float32),
            pltpu.VMEM_SHARED((NS, 2, CHUNK, D), jnp.float32),
        ]
        + [pltpu.SemaphoreType.DMA] * (2 * NBUF),
    )(x2)


def kernel(x):
    B, L, F = x.shape
    x2 = x.reshape(B * L // 2, 2 * F)
    out = _shuffle_sc(x2)
    return out.reshape(B, L, F)


# speedup vs baseline: 1.0206x; 1.0206x over previous
import jax
import jax.numpy as jnp
from jax import lax
from jax.experimental import pallas as pl
from jax.experimental.pallas import tpu as pltpu
from jax.experimental.pallas import tpu_sc as plsc

NC, NS = 2, 16
NW = NC * NS
ROWS = 16384
D = 1024
RPW = ROWS // NW
CHUNK = 32
NCHUNK = RPW // CHUNK
NBUF = 3
LEAD = NBUF - 2


def _sc_body(x2_hbm, out_hbm, sbuf, *sems):
    sid = lax.axis_index("s")
    wid = sid * NC + lax.axis_index("c")
    buf = sbuf.at[sid]
    b = wid // 8
    h = (wid % 8) // 4
    p = wid % 4
    src_row0 = b * 2048 + p * RPW
    src_col0 = h * D
    dst_row0 = wid * RPW

    gsems, psems = sems[:NBUF], sems[NBUF:]

    def gather(k):
        s = k % NBUF
        return pltpu.async_copy(
            x2_hbm.at[pl.ds(src_row0 + k * CHUNK, CHUNK), pl.ds(src_col0, D)],
            buf.at[s], gsems[s])

    def put(k):
        s = k % NBUF
        return pltpu.async_copy(
            buf.at[s], out_hbm.at[pl.ds(dst_row0 + k * CHUNK, CHUNK)],
            psems[s])

    gd = [None] * NCHUNK
    pd = [None] * NCHUNK
    for k in range(min(LEAD, NCHUNK)):
        gd[k] = gather(k)
    for k in range(NCHUNK):
        j = k + LEAD
        if j < NCHUNK:
            if j >= NBUF:
                pd[j - NBUF].wait()
            gd[j] = gather(j)
        gd[k].wait()
        pd[k] = put(k)
    for k in range(max(0, NCHUNK - NBUF), NCHUNK):
        pd[k].wait()


def _shuffle_sc(x2):
    mesh = plsc.VectorSubcoreMesh(core_axis_name="c", subcore_axis_name="s")
    return pl.kernel(
        _sc_body,
        out_type=jax.ShapeDtypeStruct((ROWS, D), jnp.float32),
        mesh=mesh,
        scratch_types=[pltpu.VMEM_SHARED((NS, NBUF, CHUNK, D), jnp.float32)]
        + [pltpu.SemaphoreType.DMA] * (2 * NBUF),
    )(x2)


def kernel(x):
    B, L, F = x.shape
    x2 = x.reshape(B * L // 2, 2 * F)
    out = _shuffle_sc(x2)
    return out.reshape(B, L, F)
